# baseline (speedup 1.0000x reference)
import functools

import jax
import jax.numpy as jnp
from jax import lax
from jax.experimental import pallas as pl
from jax.experimental.pallas import tpu as pltpu
from jax.experimental.pallas import tpu_sc as plsc

T, D, DFF = 2048, 2048, 1024
NE = 8
NTOT = 16
SCALING = 2.5

_RT = 512
_B = 256
_NB = 24
_APAD = _NB * _B
_XS_ROWS = 6400
_NW = 32
_NA = 2 * T


def _router_body(x_ref, wr_ref, b_ref, zout_ref, id1_ref, id2_ref,
                 w1_ref, w2_ref, h1_ref, h2_ref):
    x = x_ref[...]
    logits = lax.dot_general(
        x.astype(jnp.bfloat16), wr_ref[...].astype(jnp.bfloat16),
        (((1,), (1,)), ((), ())), preferred_element_type=jnp.float32)
    m = jnp.max(logits, axis=1, keepdims=True)
    ex = jnp.exp(logits - m)
    p = ex / jnp.sum(ex, axis=1, keepdims=True)
    s = p + b_ref[...]
    iota = lax.broadcasted_iota(jnp.int32, s.shape, 1)
    m1 = jnp.max(s, axis=1, keepdims=True)
    i1 = jnp.min(jnp.where(s == m1, iota, NTOT), axis=1, keepdims=True)
    s2 = jnp.where(iota == i1, -jnp.inf, s)
    m2 = jnp.max(s2, axis=1, keepdims=True)
    i2 = jnp.min(jnp.where(s2 == m2, iota, NTOT), axis=1, keepdims=True)
    w1 = m1 * SCALING
    w2 = m2 * SCALING
    zsum = jnp.where(i1 >= NE, w1, 0.0) + jnp.where(i2 >= NE, w2, 0.0)
    zout_ref[...] = x * zsum
    id1_ref[...] = i1
    id2_ref[...] = i2
    w1_ref[...] = w1
    w2_ref[...] = w2
    oh1 = jnp.where(iota == i1, 1, 0)
    oh2 = jnp.where(iota == i2, 1, 0)
    for c in range(_RT // 128):
        h1_ref[0, c] = jnp.sum(oh1[c * 128:(c + 1) * 128, :], axis=0)
        h2_ref[0, c] = jnp.sum(oh2[c * 128:(c + 1) * 128, :], axis=0)


def _dispatch_body(ids_hbm, hists_hbm, x_hbm, pos_hbm, xs_hbm,
                   idsbuf, histsv, posbuf, xrow, sem):
    wid = lax.axis_index("s") * 2 + lax.axis_index("c")
    iota16 = lax.broadcasted_iota(jnp.int32, (16,), 0)
    pltpu.sync_copy(ids_hbm.at[pl.ds(wid * 128, 128)], idsbuf)
    pltpu.sync_copy(hists_hbm, histsv)
    widv = lax.broadcast_in_dim(wid, (16,), ())
    pre = jnp.zeros((16,), jnp.int32)
    tot = jnp.zeros((16,), jnp.int32)
    for w2 in range(_NW):
        row = histsv[w2]
        pre = pre + jnp.where(jnp.full((16,), w2, jnp.int32) < widv, row, 0)
        tot = tot + row
    tot = jnp.where(iota16 < NE, tot, 0)
    pre = jnp.where(iota16 < NE, pre, 0)
    cap = jnp.bitwise_and(tot + (_B - 1), -_B)
    seg = plsc.cumsum(cap) - cap
    start = seg + pre
    for g in range(8):
        v = idsbuf[pl.ds(g * 16, 16)]
        valid = v < NE
        slot = jnp.zeros((16,), jnp.int32)
        for e in range(NE):
            msk = v == e
            rank = plsc.cumsum(jnp.where(msk, 1, 0)) - 1
            base = jnp.sum(jnp.where(iota16 == e, start, 0))
            basev = lax.broadcast_in_dim(base, (16,), ())
            slot = jnp.where(msk, basev + rank, slot)
            cnt = plsc.all_reduce_population_count(msk)
            start = start + jnp.where(iota16 == e, cnt, 0)
        sidx = jnp.where(valid, slot, _APAD + iota16)
        posbuf[pl.ds(g * 16, 16)] = jnp.where(valid, slot, 0)
        t0 = lax.rem(wid, 16) * 128 + g * 16
        pltpu.sync_copy(x_hbm.at[pl.ds(t0, 16)], xrow)
        pltpu.async_copy(xrow, xs_hbm.at[sidx], sem).wait()
    pltpu.sync_copy(posbuf, pos_hbm.at[pl.ds(wid * 128, 128)])


def _gemm_body(fl_ref, xi_ref, wi_ref, xs_ref, wg_ref, wu_ref, wd_ref, y_ref):
    g = pl.program_id(0)

    @pl.when(fl_ref[g] == 1)
    def _():
        xb = xs_ref[...].astype(jnp.bfloat16)
        gate = lax.dot_general(xb, wg_ref[0], (((1,), (1,)), ((), ())),
                               preferred_element_type=jnp.float32)
        up = lax.dot_general(xb, wu_ref[0], (((1,), (1,)), ((), ())),
                             preferred_element_type=jnp.float32)
        act = gate * (1.0 / (1.0 + jnp.exp(-gate))) * up
        y_ref[...] = lax.dot_general(
            act.astype(jnp.bfloat16), wd_ref[0], (((1,), (1,)), ((), ())),
            preferred_element_type=jnp.float32)


def _gather_body(y_hbm, pos_hbm, yd0_hbm, yd1_hbm,
                 p0b, p1b, y0v, y1v, sem0, sem1):
    wid = lax.axis_index("s") * 2 + lax.axis_index("c")
    tb = wid * 64
    pltpu.sync_copy(pos_hbm.at[pl.ds(tb, 64)], p0b)
    pltpu.sync_copy(pos_hbm.at[pl.ds(T + tb, 64)], p1b)
    for c in range(4):
        p0 = p0b[pl.ds(c * 16, 16)]
        p1 = p1b[pl.ds(c * 16, 16)]
        g0 = pltpu.async_copy(y_hbm.at[p0], y0v, sem0)
        g1 = pltpu.async_copy(y_hbm.at[p1], y1v, sem1)
        g0.wait()
        g1.wait()
        d0 = pltpu.async_copy(y0v, yd0_hbm.at[pl.ds(tb + c * 16, 16)], sem0)
        d1 = pltpu.async_copy(y1v, yd1_hbm.at[pl.ds(tb + c * 16, 16)], sem1)
        d0.wait()
        d1.wait()


def _final_body(z_ref, y0_ref, y1_ref, w1_ref, w2_ref, id1_ref, id2_ref,
                out_ref):
    zero = jnp.zeros_like(z_ref[...])
    acc = z_ref[...]
    acc = acc + jnp.where(id1_ref[...] < NE, w1_ref[...] * y0_ref[...], zero)
    acc = acc + jnp.where(id2_ref[...] < NE, w2_ref[...] * y1_ref[...], zero)
    out_ref[...] = acc


def kernel(hidden_states, W_router, correction_bias, w_gate, w_up, w_down):
    f32 = jnp.float32
    i32 = jnp.int32
    nrt = T // _RT
    zout, id1, id2, w1, w2, h1, h2 = pl.pallas_call(
        _router_body,
        grid=(nrt,),
        in_specs=[
            pl.BlockSpec((_RT, D), lambda t: (t, 0)),
            pl.BlockSpec((NTOT, D), lambda t: (0, 0)),
            pl.BlockSpec((1, NTOT), lambda t: (0, 0)),
        ],
        out_specs=[
            pl.BlockSpec((_RT, D), lambda t: (t, 0)),
            pl.BlockSpec((_RT, 1), lambda t: (t, 0)),
            pl.BlockSpec((_RT, 1), lambda t: (t, 0)),
            pl.BlockSpec((_RT, 1), lambda t: (t, 0)),
            pl.BlockSpec((_RT, 1), lambda t: (t, 0)),
            pl.BlockSpec((1, _RT // 128, NTOT), lambda t: (t, 0, 0)),
            pl.BlockSpec((1, _RT // 128, NTOT), lambda t: (t, 0, 0)),
        ],
        out_shape=[
            jax.ShapeDtypeStruct((T, D), f32),
            jax.ShapeDtypeStruct((T, 1), i32),
            jax.ShapeDtypeStruct((T, 1), i32),
            jax.ShapeDtypeStruct((T, 1), f32),
            jax.ShapeDtypeStruct((T, 1), f32),
            jax.ShapeDtypeStruct((nrt, _RT // 128, NTOT), i32),
            jax.ShapeDtypeStruct((nrt, _RT // 128, NTOT), i32),
        ],
    )(hidden_states, W_router, correction_bias.reshape(1, NTOT))

    ids = jnp.concatenate([id1, id2], axis=0)[:, 0]
    hists = jnp.concatenate([h1.reshape(16, NTOT),
                             h2.reshape(16, NTOT)], axis=0)

    mesh = plsc.VectorSubcoreMesh(core_axis_name="c", subcore_axis_name="s")
    pos, xs = pl.kernel(
        _dispatch_body,
        compiler_params=pltpu.CompilerParams(needs_layout_passes=False),
        out_type=(
            jax.ShapeDtypeStruct((_NA,), i32),
            jax.ShapeDtypeStruct((_XS_ROWS, D), f32),
        ),
        mesh=mesh,
        scratch_types=[
            pltpu.VMEM((128,), i32),
            pltpu.VMEM((_NW, 16), i32),
            pltpu.VMEM((128,), i32),
            pltpu.VMEM((16, D), f32),
            pltpu.SemaphoreType.DMA,
        ],
    )(ids, hists, hidden_states)

    counts = jnp.sum(hists, axis=0)[:NE]
    nbe = (counts + (_B - 1)) // _B
    bounds = jnp.cumsum(nbe)
    g_idx = jnp.arange(_NB, dtype=i32)
    blk_e = jnp.sum((g_idx[:, None] >= bounds[None, :]).astype(i32), axis=1)
    total = bounds[NE - 1]
    last = jnp.maximum(total - 1, 0)
    blk_e_c = jnp.minimum(blk_e, NE - 1)
    last_e = jnp.take(blk_e_c, last)
    active = (g_idx < total).astype(i32)
    xi = jnp.where(active == 1, g_idx, last).astype(i32)
    wi = jnp.where(active == 1, blk_e_c, last_e).astype(i32)

    wgb = w_gate.astype(jnp.bfloat16)
    wub = w_up.astype(jnp.bfloat16)
    wdb = w_down.astype(jnp.bfloat16)
    y = pl.pallas_call(
        _gemm_body,
        grid_spec=pltpu.PrefetchScalarGridSpec(
            num_scalar_prefetch=3,
            grid=(_NB,),
            in_specs=[
                pl.BlockSpec((_B, D), lambda g, fl, xi, wi: (xi[g], 0)),
                pl.BlockSpec((1, DFF, D), lambda g, fl, xi, wi: (wi[g], 0, 0)),
                pl.BlockSpec((1, DFF, D), lambda g, fl, xi, wi: (wi[g], 0, 0)),
                pl.BlockSpec((1, D, DFF), lambda g, fl, xi, wi: (wi[g], 0, 0)),
            ],
            out_specs=pl.BlockSpec((_B, D), lambda g, fl, xi, wi: (xi[g], 0)),
        ),
        out_shape=jax.ShapeDtypeStruct((_APAD, D), f32),
    )(active, xi, wi, xs, wgb, wub, wdb)

    yd0, yd1 = pl.kernel(
        _gather_body,
        compiler_params=pltpu.CompilerParams(needs_layout_passes=False),
        out_type=(
            jax.ShapeDtypeStruct((T, D), f32),
            jax.ShapeDtypeStruct((T, D), f32),
        ),
        mesh=plsc.VectorSubcoreMesh(core_axis_name="c", subcore_axis_name="s"),
        scratch_types=[
            pltpu.VMEM((64,), i32),
            pltpu.VMEM((64,), i32),
            pltpu.VMEM((16, D), f32),
            pltpu.VMEM((16, D), f32),
            pltpu.SemaphoreType.DMA,
            pltpu.SemaphoreType.DMA,
        ],
    )(y, pos)

    out = pl.pallas_call(
        _final_body,
        grid=(T // _RT,),
        in_specs=[
            pl.BlockSpec((_RT, D), lambda t: (t, 0)),
            pl.BlockSpec((_RT, D), lambda t: (t, 0)),
            pl.BlockSpec((_RT, D), lambda t: (t, 0)),
            pl.BlockSpec((_RT, 1), lambda t: (t, 0)),
            pl.BlockSpec((_RT, 1), lambda t: (t, 0)),
            pl.BlockSpec((_RT, 1), lambda t: (t, 0)),
            pl.BlockSpec((_RT, 1), lambda t: (t, 0)),
        ],
        out_specs=pl.BlockSpec((_RT, D), lambda t: (t, 0)),
        out_shape=jax.ShapeDtypeStruct((T, D), f32),
    )(zout, yd0, yd1, w1, w2, id1, id2)
    return out


# speedup vs baseline: 1.2461x; 1.2461x over previous
import jax
import jax.numpy as jnp
from jax import lax
from jax.experimental import pallas as pl

T, D, DFF = 2048, 2048, 1024
NE = 8
NTOT = 16
SCALING = 2.5

_RT = 512
_BT = 512


def _router_body(x_ref, wr_ref, b_ref, comb_ref, zout_ref):
    x = x_ref[...]
    logits = lax.dot_general(
        x.astype(jnp.bfloat16), wr_ref[...].astype(jnp.bfloat16),
        (((1,), (1,)), ((), ())), preferred_element_type=jnp.float32)
    m = jnp.max(logits, axis=1, keepdims=True)
    ex = jnp.exp(logits - m)
    p = ex / jnp.sum(ex, axis=1, keepdims=True)
    s = p + b_ref[...]
    iota = lax.broadcasted_iota(jnp.int32, s.shape, 1)
    m1 = jnp.max(s, axis=1, keepdims=True)
    i1 = jnp.min(jnp.where(s == m1, iota, NTOT), axis=1, keepdims=True)
    s2 = jnp.where(iota == i1, -jnp.inf, s)
    m2 = jnp.max(s2, axis=1, keepdims=True)
    i2 = jnp.min(jnp.where(s2 == m2, iota, NTOT), axis=1, keepdims=True)
    w1 = m1 * SCALING
    w2 = m2 * SCALING
    zsum = jnp.where(i1 >= NE, w1, 0.0) + jnp.where(i2 >= NE, w2, 0.0)
    iota8 = lax.broadcasted_iota(jnp.int32, (x.shape[0], NE), 1)
    comb_ref[...] = (jnp.where(iota8 == i1, w1, 0.0)
                     + jnp.where(iota8 == i2, w2, 0.0))
    zout_ref[...] = x * zsum


def _dense_body(xb_ref, comb_ref, zout_ref, wg_ref, wu_ref, wd_ref, out_ref):
    e = pl.program_id(0)
    t = pl.program_id(1)
    rows = pl.ds(t * _BT, _BT)

    @pl.when(e == 0)
    def _():
        out_ref[rows, :] = zout_ref[...]

    xb = xb_ref[...]
    gate = lax.dot_general(xb, wg_ref[0], (((1,), (1,)), ((), ())),
                           preferred_element_type=jnp.float32)
    up = lax.dot_general(xb, wu_ref[0], (((1,), (1,)), ((), ())),
                         preferred_element_type=jnp.float32)
    act = gate * (1.0 / (1.0 + jnp.exp(-gate))) * up
    iota8 = lax.broadcasted_iota(jnp.int32, comb_ref.shape, 1)
    ce = jnp.sum(jnp.where(iota8 == e, comb_ref[...], 0.0), axis=1,
                 keepdims=True)
    act = act * ce
    out_ref[rows, :] += lax.dot_general(
        act.astype(jnp.bfloat16), wd_ref[0], (((1,), (1,)), ((), ())),
        preferred_element_type=jnp.float32)


def kernel(hidden_states, W_router, correction_bias, w_gate, w_up, w_down):
    f32 = jnp.float32
    comb, zout = pl.pallas_call(
        _router_body,
        grid=(T // _RT,),
        in_specs=[
            pl.BlockSpec((_RT, D), lambda t: (t, 0)),
            pl.BlockSpec((NTOT, D), lambda t: (0, 0)),
            pl.BlockSpec((1, NTOT), lambda t: (0, 0)),
        ],
        out_specs=[
            pl.BlockSpec((_RT, NE), lambda t: (t, 0)),
            pl.BlockSpec((_RT, D), lambda t: (t, 0)),
        ],
        out_shape=[
            jax.ShapeDtypeStruct((T, NE), f32),
            jax.ShapeDtypeStruct((T, D), f32),
        ],
    )(hidden_states, W_router, correction_bias.reshape(1, NTOT))

    xb = hidden_states.astype(jnp.bfloat16)
    wgb = w_gate.astype(jnp.bfloat16)
    wub = w_up.astype(jnp.bfloat16)
    wdb = w_down.astype(jnp.bfloat16)

    out = pl.pallas_call(
        _dense_body,
        grid=(NE, T // _BT),
        in_specs=[
            pl.BlockSpec((_BT, D), lambda e, t: (t, 0)),
            pl.BlockSpec((_BT, NE), lambda e, t: (t, 0)),
            pl.BlockSpec((_BT, D), lambda e, t: (t, 0)),
            pl.BlockSpec((1, DFF, D), lambda e, t: (e, 0, 0)),
            pl.BlockSpec((1, DFF, D), lambda e, t: (e, 0, 0)),
            pl.BlockSpec((1, D, DFF), lambda e, t: (e, 0, 0)),
        ],
        out_specs=pl.BlockSpec((T, D), lambda e, t: (0, 0)),
        out_shape=jax.ShapeDtypeStruct((T, D), f32),
    )(xb, comb, zout, wgb, wub, wdb)
    return out
